# baseline (speedup 1.0000x reference)
import functools

import jax
import jax.numpy as jnp
from jax import lax
from jax.experimental import pallas as pl
from jax.experimental.pallas import tpu as pltpu
from jax.experimental.pallas import tpu_sc as plsc

N = 10000
E = 160000
D_FEAT = 256
F = 64

NC, NS = 2, 16
NPAD = 10112
EP = 163840
K = 128
EPT = EP // (NC * NS)
NCHUNK = EPT // K
RPT = NPAD // NS
WROW = 80
WACC = 80

_f32 = jnp.float32


def _tc_transform_body(x_ref, w_ref, p_ref, pt_ref, ts_ref, tt_ref):
    xt = jnp.dot(x_ref[...], w_ref[...], preferred_element_type=_f32)
    ts_ref[0:N] = jnp.dot(xt, p_ref[...], preferred_element_type=_f32)
    ts_ref[N:NPAD] = jnp.zeros((NPAD - N, WROW), _f32)
    tt_ref[0:N] = jnp.dot(xt, pt_ref[...], preferred_element_type=_f32)
    tt_ref[N:NPAD] = jnp.zeros((NPAD - N, 16), _f32)


_tc_transform = pl.pallas_call(
    _tc_transform_body,
    out_shape=(
        jax.ShapeDtypeStruct((NPAD, WROW), _f32),
        jax.ShapeDtypeStruct((NPAD, 16), _f32),
    ),
)


def _tc_mid_body(acc_ref, w_ref, p_ref, pt_ref, r_ref, ts_ref, tt_ref):
    s = acc_ref[0] + acc_ref[1]
    num = s[:, 16:80]
    den = jnp.dot(s[:, 0:8], r_ref[...], preferred_element_type=_f32)
    x1 = jnp.maximum(num / jnp.maximum(den, 1e-20), 0.0)
    xt = jnp.dot(x1, w_ref[...], preferred_element_type=_f32)
    ts_ref[...] = jnp.dot(xt, p_ref[...], preferred_element_type=_f32)
    tt_ref[...] = jnp.dot(xt, pt_ref[...], preferred_element_type=_f32)


_tc_mid = pl.pallas_call(
    _tc_mid_body,
    out_shape=(
        jax.ShapeDtypeStruct((NPAD, WROW), _f32),
        jax.ShapeDtypeStruct((NPAD, 16), _f32),
    ),
)


def _tc_final_body(acc_ref, out_ref):
    s = acc_ref[0][:N] + acc_ref[1][:N]
    num = s[:, 16:80]
    den = s[:, 0:1]
    out_ref[...] = jnp.maximum(num / jnp.maximum(den, 1e-20), 0.0)


_tc_final = pl.pallas_call(
    _tc_final_body,
    out_shape=jax.ShapeDtypeStruct((N, F), _f32),
)


def _sc_edge_body(shift, t_hbm, s_hbm, tt_hbm, ts_hbm, zz_hbm, out_hbm,
                  idx_s, sidx, g_t, g_s, w, acc,
                  gsem0, gsem1, ssem0, ssem1, ssem2, ssem3):
    c = lax.axis_index("c")
    sid = lax.axis_index("s")
    r0 = sid * RPT
    gsem = (gsem0, gsem1)
    ssem = (ssem0, ssem1, ssem2, ssem3)

    wid = c * NS + sid
    base0 = wid * EPT
    iota = lax.iota(jnp.int32, 16)
    cols = [lax.shift_right_logical(iota + 16 * k, shift) for k in range(4)]

    def issue(ci, b, t4):
        base = pl.multiple_of(base0 + ci * K, 8)
        pltpu.sync_copy(t_hbm.at[pl.ds(base, K)], sidx.at[t4])
        pltpu.sync_copy(s_hbm.at[pl.ds(base, K)], idx_s.at[b])
        pltpu.async_copy(tt_hbm.at[sidx.at[t4]], g_t.at[b], gsem[b])
        pltpu.async_copy(ts_hbm.at[idx_s.at[b]], g_s.at[b], gsem[b])

    issue(0, 0, 0)
    pltpu.sync_copy(zz_hbm.at[pl.ds(r0, RPT)], acc.at[pl.ds(r0, RPT)])
    plsc.subcore_barrier()

    def drain(slot):
        pltpu.make_async_copy(w.at[slot], acc.at[sidx.at[slot]],
                              ssem[slot]).wait()

    @pl.loop(0, NCHUNK, step=4)
    def _outer(ci):
        for j in range(4):
            b = j % 2
            cur = ci + j
            @pl.when(cur >= 3)
            def _():
                drain((j + 1) % 4)

            @pl.when(cur + 1 < NCHUNK)
            def _():
                issue(cur + 1, 1 - b, (j + 1) % 4)

            pltpu.make_async_copy(tt_hbm.at[sidx.at[j]], g_t.at[b],
                                  gsem[b]).wait()
            pltpu.make_async_copy(ts_hbm.at[idx_s.at[b]], g_s.at[b],
                                  gsem[b]).wait()

            @pl.loop(0, K)
            def _edge(e):
                sc = g_t[b, e, :] + g_s[b, e, 0:16]
                sc = jnp.where(sc >= 0.0, sc, sc * 0.2)
                p = jnp.exp(jnp.minimum(jnp.maximum(sc, -2.0), 2.0))
                w[j, e, 0:16] = p
                for k in range(4):
                    pd = p.at[cols[k]].get(mode="promise_in_bounds")
                    xv = g_s[b, e, pl.ds(16 + 16 * k, 16)]
                    w[j, e, pl.ds(16 + 16 * k, 16)] = pd * xv

            pltpu.async_copy(w.at[j], acc.at[sidx.at[j]], ssem[j], add=True)

    drain(1)
    drain(2)
    drain(3)

    plsc.subcore_barrier()
    pltpu.sync_copy(acc.at[pl.ds(r0, RPT)], out_hbm.at[c, pl.ds(r0, RPT)])


def _make_sc_edge(shift):
    return pl.kernel(
        functools.partial(_sc_edge_body, shift),
        out_type=jax.ShapeDtypeStruct((NC, NPAD, WACC), _f32),
        mesh=plsc.VectorSubcoreMesh(core_axis_name="c", subcore_axis_name="s"),
        scratch_types=[
            pltpu.VMEM((2, K), jnp.int32),
            pltpu.VMEM((4, K), jnp.int32),
            pltpu.VMEM((2, K, 16), _f32),
            pltpu.VMEM((2, K, WROW), _f32),
            pltpu.VMEM((4, K, WACC), _f32),
            pltpu.VMEM_SHARED((NPAD, WACC), _f32),
            pltpu.SemaphoreType.DMA,
            pltpu.SemaphoreType.DMA,
            pltpu.SemaphoreType.DMA,
            pltpu.SemaphoreType.DMA,
            pltpu.SemaphoreType.DMA,
            pltpu.SemaphoreType.DMA,
        ],
        compiler_params=pltpu.CompilerParams(use_tc_tiling_on_sc=False,
                                             disable_bounds_checks=True,
                                             skip_device_barrier=True),
    )


_sc_edge_l1 = _make_sc_edge(3)
_sc_edge_l2 = _make_sc_edge(6)


def _block_diag_att(a_half):
    h, u = a_half.shape
    mask = jnp.kron(jnp.eye(h, dtype=_f32), jnp.ones((u, 1), _f32))
    return mask * jnp.tile(a_half.T, (h, 1))


def kernel(node_states, edges, W1, A1, W2, A2):
    h1, _, u1 = W1.shape
    w1r = W1.transpose(1, 0, 2).reshape(D_FEAT, F)
    a1t = _block_diag_att(A1[:, :u1, 0])
    a1s = _block_diag_att(A1[:, u1:, 0])
    z8 = jnp.zeros((F, 8), _f32)
    eye = jnp.eye(F, dtype=_f32)
    p1 = jnp.concatenate([a1s, z8, eye], axis=1)
    pt1 = jnp.concatenate([a1t, z8], axis=1)

    w2r = W2[0]
    z15 = jnp.zeros((F, 15), _f32)
    p2 = jnp.concatenate([A2[0, F:, :], z15, eye], axis=1)
    pt2 = jnp.concatenate([A2[0, :F, :], z15], axis=1)
    rmat = jnp.kron(jnp.eye(8, dtype=_f32), jnp.ones((1, 8), _f32))

    ntile = NC * NS
    ppt = (EP - E) // ntile
    rpt_e = E // ntile
    pad_i = jnp.arange(ppt, dtype=jnp.int32)
    t_pad = jnp.broadcast_to(N + (pad_i % 16), (ntile, ppt))
    s_pad = jnp.broadcast_to(pad_i * 73 % N, (ntile, ppt))
    t_full = jnp.concatenate(
        [edges[:, 0].reshape(ntile, rpt_e), t_pad], axis=1).reshape(EP)
    s_full = jnp.concatenate(
        [edges[:, 1].reshape(ntile, rpt_e), s_pad], axis=1).reshape(EP)

    zz = jnp.zeros((NPAD, WACC), _f32)

    ts1, tt1 = _tc_transform(node_states, w1r, p1, pt1)
    acc1 = _sc_edge_l1(t_full, s_full, tt1, ts1, zz)
    ts2, tt2 = _tc_mid(acc1, w2r, p2, pt2, rmat)
    acc2 = _sc_edge_l2(t_full, s_full, tt2, ts2, zz)
    return _tc_final(acc2)


# speedup vs baseline: 1.0012x; 1.0012x over previous
import functools

import jax
import jax.numpy as jnp
from jax import lax
from jax.experimental import pallas as pl
from jax.experimental.pallas import tpu as pltpu
from jax.experimental.pallas import tpu_sc as plsc

N = 10000
E = 160000
D_FEAT = 256
F = 64

NC, NS = 2, 16
NPAD = 10112
EP = 163840
K = 128
EPT = EP // (NC * NS)
NCHUNK = EPT // K
RPT = NPAD // NS
WROW = 80
WACC = 80

_f32 = jnp.float32


def _tc_transform_body(x_ref, w_ref, p_ref, pt_ref, ts_ref, tt_ref):
    xt = jnp.dot(x_ref[...], w_ref[...], preferred_element_type=_f32)
    ts_ref[0:N] = jnp.dot(xt, p_ref[...], preferred_element_type=_f32)
    ts_ref[N:NPAD] = jnp.zeros((NPAD - N, WROW), _f32)
    tt_ref[0:N] = jnp.dot(xt, pt_ref[...], preferred_element_type=_f32)
    tt_ref[N:NPAD] = jnp.zeros((NPAD - N, 16), _f32)


_tc_transform = pl.pallas_call(
    _tc_transform_body,
    out_shape=(
        jax.ShapeDtypeStruct((NPAD, WROW), _f32),
        jax.ShapeDtypeStruct((NPAD, 16), _f32),
    ),
)


def _tc_mid_body(acc_ref, w_ref, p_ref, pt_ref, r_ref, ts_ref, tt_ref):
    s = acc_ref[0] + acc_ref[1]
    num = s[:, 16:80]
    den = jnp.dot(s[:, 0:8], r_ref[...], preferred_element_type=_f32)
    x1 = jnp.maximum(num / jnp.maximum(den, 1e-20), 0.0)
    xt = jnp.dot(x1, w_ref[...], preferred_element_type=_f32)
    ts_ref[...] = jnp.dot(xt, p_ref[...], preferred_element_type=_f32)
    tt_ref[...] = jnp.dot(xt, pt_ref[...], preferred_element_type=_f32)


_tc_mid = pl.pallas_call(
    _tc_mid_body,
    out_shape=(
        jax.ShapeDtypeStruct((NPAD, WROW), _f32),
        jax.ShapeDtypeStruct((NPAD, 16), _f32),
    ),
)


def _tc_final_body(acc_ref, out_ref):
    s = acc_ref[0][:N] + acc_ref[1][:N]
    num = s[:, 16:80]
    den = s[:, 0:1]
    out_ref[...] = jnp.maximum(num / jnp.maximum(den, 1e-20), 0.0)


_tc_final = pl.pallas_call(
    _tc_final_body,
    out_shape=jax.ShapeDtypeStruct((N, F), _f32),
)


def _sc_edge_body(shift, t_hbm, s_hbm, tt_hbm, ts_hbm, zz_hbm, out_hbm,
                  idx_s, sidx, g_t, g_s, w, acc,
                  gsem0, gsem1, ssem0, ssem1, ssem2, ssem3):
    c = lax.axis_index("c")
    sid = lax.axis_index("s")
    r0 = sid * RPT
    gsem = (gsem0, gsem1)
    ssem = (ssem0, ssem1, ssem2, ssem3)

    wid = c * NS + sid
    base0 = wid * EPT
    iota = lax.iota(jnp.int32, 16)
    cols = [lax.shift_right_logical(iota + 16 * k, shift) for k in range(4)]

    def issue(ci, b, t4):
        base = pl.multiple_of(base0 + ci * K, 8)
        pltpu.sync_copy(t_hbm.at[pl.ds(base, K)], sidx.at[t4])
        pltpu.sync_copy(s_hbm.at[pl.ds(base, K)], idx_s.at[b])
        pltpu.async_copy(tt_hbm.at[sidx.at[t4]], g_t.at[b], gsem[b])
        pltpu.async_copy(ts_hbm.at[idx_s.at[b]], g_s.at[b], gsem[b])

    issue(0, 0, 0)
    pltpu.sync_copy(zz_hbm.at[pl.ds(r0, RPT)], acc.at[pl.ds(r0, RPT)])
    plsc.subcore_barrier()

    def drain(slot):
        pltpu.make_async_copy(w.at[slot], acc.at[sidx.at[slot]],
                              ssem[slot]).wait()

    @pl.loop(0, NCHUNK, step=4)
    def _outer(ci):
        for j in range(4):
            b = j % 2
            cur = ci + j
            @pl.when(cur >= 3)
            def _():
                drain((j + 1) % 4)

            @pl.when(cur + 1 < NCHUNK)
            def _():
                issue(cur + 1, 1 - b, (j + 1) % 4)

            pltpu.make_async_copy(tt_hbm.at[sidx.at[j]], g_t.at[b],
                                  gsem[b]).wait()
            pltpu.make_async_copy(ts_hbm.at[idx_s.at[b]], g_s.at[b],
                                  gsem[b]).wait()

            @pl.loop(0, K)
            def _edge(e):
                sc = g_t[b, e, :] + g_s[b, e, 0:16]
                sc = jnp.where(sc >= 0.0, sc, sc * 0.2)
                p = jnp.exp(jnp.minimum(jnp.maximum(sc, -2.0), 2.0))
                w[j, e, 0:16] = p
                for k in range(4):
                    pd = p.at[cols[k]].get(mode="promise_in_bounds")
                    xv = g_s[b, e, pl.ds(16 + 16 * k, 16)]
                    w[j, e, pl.ds(16 + 16 * k, 16)] = pd * xv

            pltpu.async_copy(w.at[j], acc.at[sidx.at[j]], ssem[j], add=True)

    drain(1)
    drain(2)
    drain(3)

    plsc.subcore_barrier()
    pltpu.sync_copy(acc.at[pl.ds(r0, RPT)], out_hbm.at[c, pl.ds(r0, RPT)])


def _make_sc_edge(shift):
    return pl.kernel(
        functools.partial(_sc_edge_body, shift),
        out_type=jax.ShapeDtypeStruct((NC, NPAD, WACC), _f32),
        mesh=plsc.VectorSubcoreMesh(core_axis_name="c", subcore_axis_name="s"),
        scratch_types=[
            pltpu.VMEM((2, K), jnp.int32),
            pltpu.VMEM((4, K), jnp.int32),
            pltpu.VMEM((2, K, 16), _f32),
            pltpu.VMEM((2, K, WROW), _f32),
            pltpu.VMEM((4, K, WACC), _f32),
            pltpu.VMEM_SHARED((NPAD, WACC), _f32),
            pltpu.SemaphoreType.DMA,
            pltpu.SemaphoreType.DMA,
            pltpu.SemaphoreType.DMA,
            pltpu.SemaphoreType.DMA,
            pltpu.SemaphoreType.DMA,
            pltpu.SemaphoreType.DMA,
        ],
        compiler_params=pltpu.CompilerParams(use_tc_tiling_on_sc=False),
    )


_sc_edge_l1 = _make_sc_edge(3)
_sc_edge_l2 = _make_sc_edge(6)


def _block_diag_att(a_half):
    h, u = a_half.shape
    mask = jnp.kron(jnp.eye(h, dtype=_f32), jnp.ones((u, 1), _f32))
    return mask * jnp.tile(a_half.T, (h, 1))


def kernel(node_states, edges, W1, A1, W2, A2):
    h1, _, u1 = W1.shape
    w1r = W1.transpose(1, 0, 2).reshape(D_FEAT, F)
    a1t = _block_diag_att(A1[:, :u1, 0])
    a1s = _block_diag_att(A1[:, u1:, 0])
    z8 = jnp.zeros((F, 8), _f32)
    eye = jnp.eye(F, dtype=_f32)
    p1 = jnp.concatenate([a1s, z8, eye], axis=1)
    pt1 = jnp.concatenate([a1t, z8], axis=1)

    w2r = W2[0]
    z15 = jnp.zeros((F, 15), _f32)
    p2 = jnp.concatenate([A2[0, F:, :], z15, eye], axis=1)
    pt2 = jnp.concatenate([A2[0, :F, :], z15], axis=1)
    rmat = jnp.kron(jnp.eye(8, dtype=_f32), jnp.ones((1, 8), _f32))

    ntile = NC * NS
    ppt = (EP - E) // ntile
    rpt_e = E // ntile
    pad_i = jnp.arange(ppt, dtype=jnp.int32)
    t_pad = jnp.broadcast_to(N + (pad_i % 16), (ntile, ppt))
    s_pad = jnp.broadcast_to(pad_i * 73 % N, (ntile, ppt))
    t_full = jnp.concatenate(
        [edges[:, 0].reshape(ntile, rpt_e), t_pad], axis=1).reshape(EP)
    s_full = jnp.concatenate(
        [edges[:, 1].reshape(ntile, rpt_e), s_pad], axis=1).reshape(EP)

    zz = jnp.zeros((NPAD, WACC), _f32)

    ts1, tt1 = _tc_transform(node_states, w1r, p1, pt1)
    acc1 = _sc_edge_l1(t_full, s_full, tt1, ts1, zz)
    ts2, tt2 = _tc_mid(acc1, w2r, p2, pt2, rmat)
    acc2 = _sc_edge_l2(t_full, s_full, tt2, ts2, zz)
    return _tc_final(acc2)
